# baseline (speedup 1.0000x reference)
import functools

import jax
import jax.numpy as jnp
from jax import lax
from jax.experimental import pallas as pl
from jax.experimental.pallas import tpu as pltpu
from jax.experimental.pallas import tpu_sc as plsc

_B = 1024
_N = 100000
_L = 16
_NW = 16
_BPW = _B // _NW


def _sc_gather_max(x_words, pos_all):
    mesh = plsc.VectorSubcoreMesh(
        core_axis_name="c", subcore_axis_name="s", num_cores=1)

    @functools.partial(
        pl.kernel,
        mesh=mesh,
        out_type=jax.ShapeDtypeStruct((_B,), jnp.float32),
        scratch_types=[
            pltpu.VMEM((2 * _BPW,), jnp.int32),
            pltpu.VMEM((2 * _BPW,), jnp.float32),
            pltpu.VMEM((_BPW,), jnp.float32),
            pltpu.SemaphoreType.DMA,
        ],
    )
    def k(x_hbm, pos_hbm, out_hbm, pos_v, gath_v, out_v, sem):
        wid = lax.axis_index("s") + lax.axis_index("c") * _NW
        base = wid * _BPW
        pltpu.sync_copy(pos_hbm.at[pl.ds(wid * 2 * _BPW, 2 * _BPW)], pos_v)
        pltpu.async_copy(x_hbm.at[pos_v], gath_v, sem).wait()
        for j in range(_BPW // _L):
            v1 = gath_v[pl.ds(j * _L, _L)]
            v2 = gath_v[pl.ds(_BPW + j * _L, _L)]
            out_v[pl.ds(j * _L, _L)] = jnp.maximum(v1, v2)
        pltpu.sync_copy(out_v, out_hbm.at[pl.ds(base, _BPW)])

    return k(x_words, pos_all)


def kernel(x):
    B, N = x.shape
    with jax.ensure_compile_time_eval():
        key = jax.random.key(42)
        k1, k2 = jax.random.split(key)
        idx1 = jax.random.randint(k1, (B,), 0, N).astype(jnp.int32)
        idx2 = jax.random.randint(k2, (B,), 0, N).astype(jnp.int32)
        b = jnp.arange(B, dtype=jnp.int32)

        def flatpos(c):
            return (((c >> 3) * 8 + (b >> 7)) * 1024
                    + (c & 7) * 128 + (b & 127))

        p1, p2 = flatpos(idx1), flatpos(idx2)
        pos_all = jnp.concatenate(
            [p1.reshape(_NW, _BPW), p2.reshape(_NW, _BPW)], axis=1
        ).reshape(-1)
    x_words = (x.T.reshape(N // 8, 8, 8, 128)
               .transpose(0, 2, 1, 3).reshape(-1))
    return _sc_gather_max(x_words, pos_all)
